# baseline (speedup 1.0000x reference)
import functools

import jax
import jax.numpy as jnp
from jax import lax
from jax.experimental import pallas as pl
from jax.experimental.pallas import tpu as pltpu
from jax.experimental.pallas import tpu_sc as plsc

B = 4096 * 200
D = 128
NW = 32
BPW = B // NW
C = 128
STEPS = BPW // C
GPH = 2
CH = GPH * C
NH = 2
NPAIR = STEPS // (GPH * NH)

_mesh = plsc.VectorSubcoreMesh(core_axis_name="c", subcore_axis_name="s")


@functools.partial(
    pl.kernel,
    mesh=_mesh,
    out_type=jax.ShapeDtypeStruct((B, D), jnp.float32),
    scratch_types=(
        [
            pltpu.VMEM((STEPS, C), jnp.int32),
            pltpu.VMEM((NH, CH, D), jnp.float32),
            pltpu.VMEM_SHARED((20, D), jnp.float32),
        ]
        + [pltpu.SemaphoreType.DMA for _ in range(2 * NH)]
    ),
)
def _emb_lookup(tokens_hbm, table_hbm, out_hbm, idx_v, rows_v, table_sh, *sems):
    gsems = sems[:NH]
    wsems = sems[NH:]
    sid = lax.axis_index("s")
    wid = sid * 2 + lax.axis_index("c")
    base = wid * BPW

    @pl.when(sid == 0)
    def _stage_table():
        pltpu.sync_copy(table_hbm, table_sh)

    pltpu.sync_copy(tokens_hbm.at[pl.ds(wid * STEPS, STEPS)], idx_v)
    plsc.subcore_barrier()

    def pair(p, carry):
        for h in range(NH):
            coff = base + (p * NH + h) * CH
            @pl.when(p > 0)
            def _wait_prev_wb(h=h, coff=coff):
                pltpu.make_async_copy(
                    rows_v.at[h], out_hbm.at[pl.ds(coff, CH)], wsems[h]
                ).wait()

            for g in range(GPH):
                pltpu.async_copy(
                    table_sh.at[idx_v.at[(p * NH + h) * GPH + g]],
                    rows_v.at[h, pl.ds(g * C, C)],
                    gsems[h],
                )
        for h in range(NH):
            coff = base + (p * NH + h) * CH
            for g in range(GPH):
                pltpu.make_async_copy(
                    table_sh.at[idx_v.at[(p * NH + h) * GPH + g]],
                    rows_v.at[h, pl.ds(g * C, C)],
                    gsems[h],
                ).wait()
            pltpu.async_copy(rows_v.at[h], out_hbm.at[pl.ds(coff, CH)], wsems[h])
        return carry

    lax.fori_loop(0, NPAIR, pair, 0)

    for h in range(NH):
        pltpu.make_async_copy(
            rows_v.at[h], out_hbm.at[pl.ds(base, CH)], wsems[h]
        ).wait()


def kernel(tokens, embedding):
    flat = tokens.reshape(-1, C).astype(jnp.int32)
    out = _emb_lookup(flat, embedding)
    return out.reshape(tokens.shape + (D,))


# speedup vs baseline: 1.4517x; 1.4517x over previous
import functools

import jax
import jax.numpy as jnp
from jax import lax
from jax.experimental import pallas as pl
from jax.experimental.pallas import tpu as pltpu
from jax.experimental.pallas import tpu_sc as plsc

B = 4096 * 200
D = 128
NW = 32
BPW = B // NW
C = 128
STEPS = BPW // C
NBUF = 4
NGROUPS = STEPS // NBUF

_mesh = plsc.VectorSubcoreMesh(core_axis_name="c", subcore_axis_name="s")


@functools.partial(
    pl.kernel,
    mesh=_mesh,
    out_type=jax.ShapeDtypeStruct((B, D), jnp.float32),
    scratch_types=(
        [
            pltpu.VMEM((STEPS, C), jnp.int32),
            pltpu.VMEM((NBUF, C, D), jnp.float32),
            pltpu.VMEM_SHARED((20, D), jnp.float32),
        ]
        + [pltpu.SemaphoreType.DMA for _ in range(2 * NBUF)]
    ),
)
def _emb_lookup(tokens_hbm, table_hbm, out_hbm, idx_v, rows_v, table_sh, *sems):
    gsems = sems[:NBUF]
    wsems = sems[NBUF:]
    sid = lax.axis_index("s")
    wid = sid * 2 + lax.axis_index("c")
    base = wid * BPW

    @pl.when(sid == 0)
    def _stage_table():
        pltpu.sync_copy(table_hbm, table_sh)

    pltpu.sync_copy(tokens_hbm.at[pl.ds(wid * STEPS, STEPS)], idx_v)
    plsc.subcore_barrier()

    def group(g, carry):
        goff = base + g * (NBUF * C)
        for b in range(NBUF):
            @pl.when(g > 0)
            def _wait_prev_wb(b=b, goff=goff):
                pltpu.make_async_copy(
                    rows_v.at[b], out_hbm.at[pl.ds(goff, C)], wsems[b]
                ).wait()

            pltpu.async_copy(
                table_sh.at[idx_v.at[g * NBUF + b]], rows_v.at[b], gsems[b]
            )
        for b in range(NBUF):
            pltpu.make_async_copy(
                table_sh.at[idx_v.at[g * NBUF + b]], rows_v.at[b], gsems[b]
            ).wait()
            pltpu.async_copy(rows_v.at[b], out_hbm.at[pl.ds(goff + b * C, C)], wsems[b])
        return carry

    lax.fori_loop(0, NGROUPS, group, 0)

    for b in range(NBUF):
        pltpu.make_async_copy(
            rows_v.at[b], out_hbm.at[pl.ds(base, C)], wsems[b]
        ).wait()


def kernel(tokens, embedding):
    flat = tokens.reshape(-1, C).astype(jnp.int32)
    out = _emb_lookup(flat, embedding)
    return out.reshape(tokens.shape + (D,))


# speedup vs baseline: 1.4559x; 1.0029x over previous
import functools

import jax
import jax.numpy as jnp
from jax import lax
from jax.experimental import pallas as pl
from jax.experimental.pallas import tpu as pltpu
from jax.experimental.pallas import tpu_sc as plsc

B = 4096 * 200
D = 128
NW = 32
BPW = B // NW
C = 64
STEPS = BPW // C
NBUF = 8
NGROUPS = STEPS // NBUF

_mesh = plsc.VectorSubcoreMesh(core_axis_name="c", subcore_axis_name="s")


@functools.partial(
    pl.kernel,
    mesh=_mesh,
    out_type=jax.ShapeDtypeStruct((B, D), jnp.float32),
    scratch_types=(
        [
            pltpu.VMEM((STEPS, C), jnp.int32),
            pltpu.VMEM((NBUF, C, D), jnp.float32),
            pltpu.VMEM_SHARED((20, D), jnp.float32),
        ]
        + [pltpu.SemaphoreType.DMA for _ in range(2 * NBUF)]
    ),
)
def _emb_lookup(tokens_hbm, table_hbm, out_hbm, idx_v, rows_v, table_sh, *sems):
    gsems = sems[:NBUF]
    wsems = sems[NBUF:]
    sid = lax.axis_index("s")
    wid = sid * 2 + lax.axis_index("c")
    base = wid * BPW

    @pl.when(sid == 0)
    def _stage_table():
        pltpu.sync_copy(table_hbm, table_sh)

    pltpu.sync_copy(tokens_hbm.at[pl.ds(wid * STEPS, STEPS)], idx_v)
    plsc.subcore_barrier()

    def group(g, carry):
        goff = base + g * (NBUF * C)
        for b in range(NBUF):
            @pl.when(g > 0)
            def _wait_prev_wb(b=b, goff=goff):
                pltpu.make_async_copy(
                    rows_v.at[b], out_hbm.at[pl.ds(goff, C)], wsems[b]
                ).wait()

            pltpu.async_copy(
                table_sh.at[idx_v.at[g * NBUF + b]], rows_v.at[b], gsems[b]
            )
        for b in range(NBUF):
            pltpu.make_async_copy(
                table_sh.at[idx_v.at[g * NBUF + b]], rows_v.at[b], gsems[b]
            ).wait()
            pltpu.async_copy(rows_v.at[b], out_hbm.at[pl.ds(goff + b * C, C)], wsems[b])
        return carry

    lax.fori_loop(0, NGROUPS, group, 0)

    for b in range(NBUF):
        pltpu.make_async_copy(
            rows_v.at[b], out_hbm.at[pl.ds(base, C)], wsems[b]
        ).wait()


def kernel(tokens, embedding):
    flat = tokens.reshape(-1, C).astype(jnp.int32)
    out = _emb_lookup(flat, embedding)
    return out.reshape(tokens.shape + (D,))


# speedup vs baseline: 1.4716x; 1.0108x over previous
import functools

import jax
import jax.numpy as jnp
from jax import lax
from jax.experimental import pallas as pl
from jax.experimental.pallas import tpu as pltpu
from jax.experimental.pallas import tpu_sc as plsc

B = 4096 * 200
D = 128
NW = 32
BPW = B // NW
C = 64
STEPS = BPW // C
NBUF = 8
NGROUPS = STEPS // NBUF

_mesh = plsc.VectorSubcoreMesh(core_axis_name="c", subcore_axis_name="s")


@functools.partial(
    pl.kernel,
    mesh=_mesh,
    out_type=jax.ShapeDtypeStruct((B, D), jnp.float32),
    scratch_types=(
        [
            pltpu.VMEM((STEPS, C), jnp.int32),
            pltpu.VMEM((NBUF, C, D), jnp.float32),
            pltpu.VMEM_SHARED((20, D), jnp.float32),
        ]
        + [pltpu.SemaphoreType.DMA for _ in range(2 * NBUF + 1)]
    ),
)
def _emb_lookup(tokens_hbm, table_hbm, out_hbm, idx_v, rows_v, table_sh, *sems):
    gsems = sems[:NBUF]
    wsems = sems[NBUF:2 * NBUF]
    isem = sems[2 * NBUF]
    sid = lax.axis_index("s")
    wid = sid * 2 + lax.axis_index("c")
    base = wid * BPW

    @pl.when(sid == 0)
    def _stage_table():
        pltpu.sync_copy(table_hbm, table_sh)

    pltpu.sync_copy(
        tokens_hbm.at[pl.ds(wid * STEPS, NBUF)], idx_v.at[pl.ds(0, NBUF)]
    )
    pltpu.async_copy(
        tokens_hbm.at[pl.ds(wid * STEPS + NBUF, STEPS - NBUF)],
        idx_v.at[pl.ds(NBUF, STEPS - NBUF)],
        isem,
    )
    plsc.subcore_barrier()

    def group(g, carry):
        goff = base + g * (NBUF * C)

        @pl.when(g == 1)
        def _wait_idx():
            pltpu.make_async_copy(
                tokens_hbm.at[pl.ds(wid * STEPS + NBUF, STEPS - NBUF)],
                idx_v.at[pl.ds(NBUF, STEPS - NBUF)],
                isem,
            ).wait()

        for b in range(NBUF):
            @pl.when(g > 0)
            def _wait_prev_wb(b=b, goff=goff):
                pltpu.make_async_copy(
                    rows_v.at[b], out_hbm.at[pl.ds(goff, C)], wsems[b]
                ).wait()

            pltpu.async_copy(
                table_sh.at[idx_v.at[g * NBUF + b]], rows_v.at[b], gsems[b]
            )
        for b in range(NBUF):
            pltpu.make_async_copy(
                table_sh.at[idx_v.at[g * NBUF + b]], rows_v.at[b], gsems[b]
            ).wait()
            pltpu.async_copy(rows_v.at[b], out_hbm.at[pl.ds(goff + b * C, C)], wsems[b])
        return carry

    lax.fori_loop(0, NGROUPS, group, 0)

    for b in range(NBUF):
        pltpu.make_async_copy(
            rows_v.at[b], out_hbm.at[pl.ds(base, C)], wsems[b]
        ).wait()


def kernel(tokens, embedding):
    flat = tokens.reshape(-1, C).astype(jnp.int32)
    out = _emb_lookup(flat, embedding)
    return out.reshape(tokens.shape + (D,))
